# baseline (speedup 1.0000x reference)
import jax
import jax.numpy as jnp
from jax import lax
from jax.experimental import pallas as pl
from jax.experimental.pallas import tpu as pltpu
from jax.experimental.pallas import tpu_sc as plsc

_ACC_K = 5
_V = 100000
_CHUNK = 20000
_NCHUNK = _V // _CHUNK
_VECS = _CHUNK // 16
_NC = 2
_NS = 16
_NW = _NC * _NS
_N = 256
_RPW = _N // _NW


def _body(logits_hbm, tgt_hbm, mask_hbm, out_hbm,
          buf0, buf1, win_v, tgt_v, mask_v, res_v, sem0, sem1):
    cid = lax.axis_index("c")
    sid = lax.axis_index("s")
    wid = sid * _NC + cid
    base = wid * _RPW

    pltpu.sync_copy(tgt_hbm.at[pl.ds(pl.multiple_of(base, 8), _RPW)],
                    tgt_v.at[pl.ds(0, _RPW)])
    pltpu.sync_copy(mask_hbm.at[pl.ds(pl.multiple_of(base, 8), _RPW)],
                    mask_v.at[pl.ds(0, _RPW)])
    tgt_all = tgt_v[...]
    mask_all = mask_v[...]

    for j in range(_RPW):
        t = tgt_all[j]
        ta = t & -8
        start = pl.multiple_of((base + j) * _V + ta, 8)
        pltpu.sync_copy(logits_hbm.at[pl.ds(start, 8)],
                        win_v.at[pl.ds(j * 8, 8)])

    lane = lax.iota(jnp.int32, 16)
    zeros_i = jnp.zeros((16,), jnp.int32)
    ones_i = jnp.ones((16,), jnp.int32)
    zeros_f = jnp.zeros((16,), jnp.float32)

    bufs = (buf0, buf1)
    sems = (sem0, sem1)
    total = _RPW * _NCHUNK

    def chunk_src(k):
        j, c = divmod(k, _NCHUNK)
        off = pl.multiple_of((base + j) * _V + c * _CHUNK, 8)
        return logits_hbm.at[pl.ds(off, _CHUNK)]

    cp = pltpu.async_copy(chunk_src(0), bufs[0], sems[0])

    res = zeros_f
    for j in range(_RPW):
        t = tgt_all[j]
        tvec = jnp.full((16,), t, jnp.int32)
        vt = plsc.load_gather(
            win_v, [jnp.full((16,), j * 8, jnp.int32) + (t & 7)])
        count = zeros_i
        for c in range(_NCHUNK):
            k = j * _NCHUNK + c
            if k + 1 < total:
                nxt = pltpu.async_copy(
                    chunk_src(k + 1), bufs[(k + 1) % 2], sems[(k + 1) % 2])
            cp.wait()
            buf = bufs[k % 2]
            cbase = c * _CHUNK

            t_rel = jnp.clip(t - cbase, 0, _CHUNK)
            nge = lax.shift_right_logical(t_rel, 4)

            accs = (count, zeros_i, zeros_i, zeros_i)

            @plsc.parallel_loop(0, nge, 1, unroll=8, carry=accs)
            def ge_loop(i, acc):
                a0, a1, a2, a3 = acc
                v = buf[pl.ds(pl.multiple_of(i * 16, 16), 16)]
                return (a1, a2, a3, a0 + jnp.where(v >= vt, ones_i, zeros_i))

            accs = ge_loop
            bidx = jnp.minimum(nge, _VECS - 1)
            v = buf[pl.ds(pl.multiple_of(bidx * 16, 16), 16)]
            pos = lane + (cbase + bidx * 16)
            mixed = (v > vt) | ((v == vt) & (pos < tvec))
            bcnt = jnp.where(mixed, ones_i, zeros_i)
            guard = jnp.full((16,), nge < _VECS)
            accs = (accs[0] + jnp.where(guard, bcnt, zeros_i),
                    accs[1], accs[2], accs[3])

            @plsc.parallel_loop(nge + 1, _VECS, 1, unroll=8, carry=accs)
            def gt_loop(i, acc):
                a0, a1, a2, a3 = acc
                v = buf[pl.ds(pl.multiple_of(i * 16, 16), 16)]
                return (a1, a2, a3, a0 + jnp.where(v > vt, ones_i, zeros_i))

            a0, a1, a2, a3 = gt_loop
            count = (a0 + a1) + (a2 + a3)
            if k + 1 < total:
                cp = nxt
        rank = jnp.sum(count)
        mf = mask_all[j].astype(jnp.float32)
        hitm = jnp.where(rank < _ACC_K, mf, jnp.float32(0.0))
        res = res + jnp.where(lane == j, jnp.full((16,), hitm), zeros_f)
        res = res + jnp.where(lane == (8 + j), jnp.full((16,), mf), zeros_f)

    res_v[...] = res
    pltpu.sync_copy(res_v, out_hbm.at[wid])


_sc_call = pl.kernel(
    _body,
    out_type=jax.ShapeDtypeStruct((_NW, 16), jnp.float32),
    mesh=plsc.VectorSubcoreMesh(core_axis_name="c", subcore_axis_name="s"),
    compiler_params=pltpu.CompilerParams(needs_layout_passes=False),
    scratch_types=[
        pltpu.VMEM((_CHUNK,), jnp.float32),
        pltpu.VMEM((_CHUNK,), jnp.float32),
        pltpu.VMEM((_RPW * 8,), jnp.float32),
        pltpu.VMEM((16,), jnp.int32),
        pltpu.VMEM((16,), jnp.int32),
        pltpu.VMEM((16,), jnp.float32),
        pltpu.SemaphoreType.DMA,
        pltpu.SemaphoreType.DMA,
    ],
)


def kernel(logits, target, mask):
    flat_logits = logits.reshape(-1)
    tgt = target.reshape(-1).astype(jnp.int32)
    msk = mask.reshape(-1).astype(jnp.int32)
    part = _sc_call(flat_logits, tgt, msk)
    counter = jnp.sum(part[:, :_RPW])
    all_counter = jnp.sum(part[:, _RPW:])
    return (counter / all_counter)[None].astype(jnp.float32)


# speedup vs baseline: 1.0599x; 1.0599x over previous
import jax
import jax.numpy as jnp
from jax import lax
from jax.experimental import pallas as pl
from jax.experimental.pallas import tpu as pltpu
from jax.experimental.pallas import tpu_sc as plsc

_ACC_K = 5
_V = 100000
_CHUNK = 20000
_HALF = _CHUNK // 2
_NCHUNK = _V // _CHUNK
_VECS = _CHUNK // 16
_NC = 2
_NS = 16
_NW = _NC * _NS
_N = 256
_RPW = _N // _NW


def _body(logits_hbm, tgt_hbm, mask_hbm, out_hbm,
          buf0, buf1, buf2, win_v, tgt_v, mask_v, res_v, *sems):
    cid = lax.axis_index("c")
    sid = lax.axis_index("s")
    wid = sid * _NC + cid
    base = wid * _RPW

    pltpu.sync_copy(tgt_hbm.at[pl.ds(pl.multiple_of(base, 8), _RPW)],
                    tgt_v.at[pl.ds(0, _RPW)])
    pltpu.sync_copy(mask_hbm.at[pl.ds(pl.multiple_of(base, 8), _RPW)],
                    mask_v.at[pl.ds(0, _RPW)])
    tgt_all = tgt_v[...]
    mask_all = mask_v[...]

    for j in range(_RPW):
        t = tgt_all[j]
        ta = t & -8
        start = pl.multiple_of((base + j) * _V + ta, 8)
        pltpu.sync_copy(logits_hbm.at[pl.ds(start, 8)],
                        win_v.at[pl.ds(j * 8, 8)])

    lane = lax.iota(jnp.int32, 16)
    zeros_i = jnp.zeros((16,), jnp.int32)
    ones_i = jnp.ones((16,), jnp.int32)
    zeros_f = jnp.zeros((16,), jnp.float32)

    bufs = (buf0, buf1, buf2)
    total = _RPW * _NCHUNK

    def issue(k):
        j, c = divmod(k, _NCHUNK)
        b = k % 3
        cps = []
        for h in range(2):
            off = pl.multiple_of(
                (base + j) * _V + c * _CHUNK + h * _HALF, 8)
            cps.append(pltpu.async_copy(
                logits_hbm.at[pl.ds(off, _HALF)],
                bufs[b].at[pl.ds(h * _HALF, _HALF)],
                sems[b * 2 + h]))
        return tuple(cps)

    pend = {0: issue(0), 1: issue(1)}

    res = zeros_f
    for j in range(_RPW):
        t = tgt_all[j]
        tvec = jnp.full((16,), t, jnp.int32)
        vt = plsc.load_gather(
            win_v, [jnp.full((16,), j * 8, jnp.int32) + (t & 7)])
        count = zeros_i
        for c in range(_NCHUNK):
            k = j * _NCHUNK + c
            if k + 2 < total:
                pend[k + 2] = issue(k + 2)
            for cp in pend.pop(k):
                cp.wait()
            buf = bufs[k % 3]
            cbase = c * _CHUNK

            t_rel = jnp.clip(t - cbase, 0, _CHUNK)
            nge = lax.shift_right_logical(t_rel, 4)

            accs = (count, zeros_i, zeros_i, zeros_i)

            @plsc.parallel_loop(0, nge, 1, unroll=8, carry=accs)
            def ge_loop(i, acc):
                a0, a1, a2, a3 = acc
                v = buf[pl.ds(pl.multiple_of(i * 16, 16), 16)]
                return (a1, a2, a3, a0 + jnp.where(v >= vt, ones_i, zeros_i))

            accs = ge_loop
            bidx = jnp.minimum(nge, _VECS - 1)
            v = buf[pl.ds(pl.multiple_of(bidx * 16, 16), 16)]
            pos = lane + (cbase + bidx * 16)
            mixed = (v > vt) | ((v == vt) & (pos < tvec))
            bcnt = jnp.where(mixed, ones_i, zeros_i)
            guard = jnp.full((16,), nge < _VECS)
            accs = (accs[0] + jnp.where(guard, bcnt, zeros_i),
                    accs[1], accs[2], accs[3])

            @plsc.parallel_loop(nge + 1, _VECS, 1, unroll=8, carry=accs)
            def gt_loop(i, acc):
                a0, a1, a2, a3 = acc
                v = buf[pl.ds(pl.multiple_of(i * 16, 16), 16)]
                return (a1, a2, a3, a0 + jnp.where(v > vt, ones_i, zeros_i))

            a0, a1, a2, a3 = gt_loop
            count = (a0 + a1) + (a2 + a3)
        rank = jnp.sum(count)
        mf = mask_all[j].astype(jnp.float32)
        hitm = jnp.where(rank < _ACC_K, mf, jnp.float32(0.0))
        res = res + jnp.where(lane == j, jnp.full((16,), hitm), zeros_f)
        res = res + jnp.where(lane == (8 + j), jnp.full((16,), mf), zeros_f)

    res_v[...] = res
    pltpu.sync_copy(res_v, out_hbm.at[wid])


_sc_call = pl.kernel(
    _body,
    out_type=jax.ShapeDtypeStruct((_NW, 16), jnp.float32),
    mesh=plsc.VectorSubcoreMesh(core_axis_name="c", subcore_axis_name="s"),
    compiler_params=pltpu.CompilerParams(needs_layout_passes=False),
    scratch_types=[
        pltpu.VMEM((_CHUNK,), jnp.float32),
        pltpu.VMEM((_CHUNK,), jnp.float32),
        pltpu.VMEM((_CHUNK,), jnp.float32),
        pltpu.VMEM((_RPW * 8,), jnp.float32),
        pltpu.VMEM((16,), jnp.int32),
        pltpu.VMEM((16,), jnp.int32),
        pltpu.VMEM((16,), jnp.float32),
        pltpu.SemaphoreType.DMA,
        pltpu.SemaphoreType.DMA,
        pltpu.SemaphoreType.DMA,
        pltpu.SemaphoreType.DMA,
        pltpu.SemaphoreType.DMA,
        pltpu.SemaphoreType.DMA,
    ],
)


def kernel(logits, target, mask):
    flat_logits = logits.reshape(-1)
    tgt = target.reshape(-1).astype(jnp.int32)
    msk = mask.reshape(-1).astype(jnp.int32)
    part = _sc_call(flat_logits, tgt, msk)
    counter = jnp.sum(part[:, :_RPW])
    all_counter = jnp.sum(part[:, _RPW:])
    return (counter / all_counter)[None].astype(jnp.float32)


# speedup vs baseline: 2.6155x; 2.4677x over previous
import jax
import jax.numpy as jnp
from jax import lax
from jax.experimental import pallas as pl
from jax.experimental.pallas import tpu as pltpu
from jax.experimental.pallas import tpu_sc as plsc

_ACC_K = 5
_V = 100000
_NC = 2
_NS = 16
_NW = _NC * _NS
_N = 256
_RPW = _N // _NW

_W = 6400
_NMAIN = 15
_WR = 3968
_CBR = _NMAIN * _W
_TAIL0 = 99968
_TVO = _V - 128


def _body(logits_hbm, tail_hbm, tgt_hbm, mask_hbm, out_hbm,
          buf0, buf1, win_v, tail_v, tgt_v, mask_v, res_v, sem0, sem1):
    cid = lax.axis_index("c")
    sid = lax.axis_index("s")
    wid = sid * _NC + cid
    base = wid * _RPW

    pltpu.sync_copy(tgt_hbm.at[pl.ds(pl.multiple_of(base, 8), _RPW)],
                    tgt_v.at[pl.ds(0, _RPW)])
    pltpu.sync_copy(mask_hbm.at[pl.ds(pl.multiple_of(base, 8), _RPW)],
                    mask_v.at[pl.ds(0, _RPW)])
    pltpu.sync_copy(tail_hbm.at[pl.ds(pl.multiple_of(base, 8), _RPW), :], tail_v)
    tgt_all = tgt_v[...]
    mask_all = mask_v[...]

    for j in range(_RPW):
        t = tgt_all[j]
        ws = pl.multiple_of(jnp.minimum(t & -128, _V - 160), 128)
        pltpu.sync_copy(logits_hbm.at[pl.ds(pl.multiple_of(base, 8), _RPW),
                                       pl.ds(ws, 128)],
                        win_v.at[pl.ds(j * _RPW, _RPW), :])

    lane = lax.iota(jnp.int32, 16)
    zeros_i = jnp.zeros((16,), jnp.int32)
    ones_i = jnp.ones((16,), jnp.int32)
    zeros_f = jnp.zeros((16,), jnp.float32)

    vts = []
    tvecs = []
    for j in range(_RPW):
        t = tgt_all[j]
        tvecs.append(jnp.full((16,), t, jnp.int32))
        ws = jnp.minimum(t & -128, _V - 160)
        colw = jnp.minimum(t - ws, 127)
        vw = plsc.load_gather(
            win_v, [jnp.full((16,), j * _RPW + j, jnp.int32),
                    jnp.full((16,), colw, jnp.int32)])
        colt = jnp.clip(t - _TVO, 0, 127)
        vt_tail = plsc.load_gather(
            tail_v, [jnp.full((16,), j, jnp.int32),
                     jnp.full((16,), colt, jnp.int32)])
        in_tail = jnp.full((16,), t >= _TAIL0)
        vts.append(jnp.where(in_tail, vt_tail, vw))

    def issue(buf, sem, cb, w):
        return pltpu.async_copy(
            logits_hbm.at[pl.ds(pl.multiple_of(base, 8), _RPW), pl.ds(cb, w)],
            buf.at[:, pl.ds(0, w)], sem)

    def wait(buf, sem, cb, w):
        pltpu.make_async_copy(
            logits_hbm.at[pl.ds(pl.multiple_of(base, 8), _RPW), pl.ds(cb, w)],
            buf.at[:, pl.ds(0, w)], sem).wait()

    def process(buf, cb, w, counts):
        nvec = w // 16
        out = []
        for j in range(_RPW):
            vt = vts[j]
            tvec = tvecs[j]
            t = tgt_all[j]
            t_rel = jnp.clip(t - cb, 0, w)
            nge = lax.shift_right_logical(t_rel, 4)

            @plsc.parallel_loop(0, nge, 1, unroll=8, carry=counts[j])
            def ge_loop(i, cnt):
                v = buf[j, pl.ds(pl.multiple_of(i * 16, 16), 16)]
                return cnt + jnp.where(v >= vt, ones_i, zeros_i)

            count = ge_loop
            bidx = jnp.minimum(nge, nvec - 1)
            v = buf[j, pl.ds(pl.multiple_of(bidx * 16, 16), 16)]
            pos = lane + (cb + bidx * 16)
            mixed = (v > vt) | ((v == vt) & (pos < tvec))
            bcnt = jnp.where(mixed, ones_i, zeros_i)
            guard = jnp.full((16,), nge < nvec)
            count = count + jnp.where(guard, bcnt, zeros_i)

            @plsc.parallel_loop(nge + 1, nvec, 1, unroll=8, carry=count)
            def gt_loop(i, cnt):
                v = buf[j, pl.ds(pl.multiple_of(i * 16, 16), 16)]
                return cnt + jnp.where(v > vt, ones_i, zeros_i)

            out.append(gt_loop)
        return tuple(out)

    issue(buf0, sem0, 0, _W)
    issue(buf1, sem1, _W, _W)

    counts0 = tuple(zeros_i for _ in range(_RPW))

    def pair_step(g, counts):
        cb = g * (2 * _W)
        wait(buf0, sem0, cb, _W)
        counts = process(buf0, cb, _W, counts)

        @pl.when(g <= _NMAIN // 2 - 1)
        def _():
            issue(buf0, sem0, cb + 2 * _W, _W)

        wait(buf1, sem1, cb + _W, _W)
        counts = process(buf1, cb + _W, _W, counts)

        @pl.when(g <= _NMAIN // 2 - 2)
        def _():
            issue(buf1, sem1, cb + 3 * _W, _W)

        @pl.when(g == _NMAIN // 2 - 1)
        def _():
            issue(buf1, sem1, _CBR, _WR)

        return counts

    counts0 = lax.fori_loop(0, _NMAIN // 2, pair_step, counts0)

    cb14 = (_NMAIN - 1) * _W
    wait(buf0, sem0, cb14, _W)
    counts0 = process(buf0, cb14, _W, counts0)
    wait(buf1, sem1, _CBR, _WR)
    counts0 = process(buf1, _CBR, _WR, counts0)

    counts = list(counts0)
    for j in range(_RPW):
        vt = vts[j]
        tvec = tvecs[j]
        for k in range(2):
            v = tail_v[j, pl.ds(96 + k * 16, 16)]
            pos = lane + (_TAIL0 + k * 16)
            mixed = (v > vt) | ((v == vt) & (pos < tvec))
            counts[j] = counts[j] + jnp.where(mixed, ones_i, zeros_i)

    res = zeros_f
    for j in range(_RPW):
        rank = jnp.sum(counts[j])
        mf = mask_all[j].astype(jnp.float32)
        hitm = jnp.where(rank < _ACC_K, mf, jnp.float32(0.0))
        res = res + jnp.where(lane == j, jnp.full((16,), hitm), zeros_f)
        res = res + jnp.where(lane == (8 + j), jnp.full((16,), mf), zeros_f)

    res_v[...] = res
    pltpu.sync_copy(res_v, out_hbm.at[pl.ds(wid * 16, 16)])


_sc_call = pl.kernel(
    _body,
    out_type=jax.ShapeDtypeStruct((_NW * 16,), jnp.float32),
    mesh=plsc.VectorSubcoreMesh(core_axis_name="c", subcore_axis_name="s"),
    compiler_params=pltpu.CompilerParams(needs_layout_passes=False),
    scratch_types=[
        pltpu.VMEM((_RPW, _W), jnp.float32),
        pltpu.VMEM((_RPW, _W), jnp.float32),
        pltpu.VMEM((_RPW * _RPW, 128), jnp.float32),
        pltpu.VMEM((_RPW, 128), jnp.float32),
        pltpu.VMEM((16,), jnp.int32),
        pltpu.VMEM((16,), jnp.int32),
        pltpu.VMEM((16,), jnp.float32),
        pltpu.SemaphoreType.DMA,
        pltpu.SemaphoreType.DMA,
    ],
)


def kernel(logits, target, mask):
    logits2d = logits.reshape(_N, _V)
    tail = logits2d[:, _TVO:]
    tgt = target.reshape(-1).astype(jnp.int32)
    msk = mask.reshape(-1).astype(jnp.int32)
    part = _sc_call(logits2d, tail, tgt, msk).reshape(_NW, 16)
    counter = jnp.sum(part[:, :_RPW])
    all_counter = jnp.sum(part[:, _RPW:])
    return (counter / all_counter)[None].astype(jnp.float32)


# speedup vs baseline: 2.8621x; 1.0942x over previous
import jax
import jax.numpy as jnp
from jax import lax
from jax.experimental import pallas as pl
from jax.experimental.pallas import tpu as pltpu
from jax.experimental.pallas import tpu_sc as plsc

_ACC_K = 5
_V = 100000
_NC = 2
_NS = 16
_NW = _NC * _NS
_N = 256
_RPW = _N // _NW

_W = 6400
_NMAIN = 15
_WR = 3968
_CBR = _NMAIN * _W
_TAIL0 = 99968
_TVO = _V - 128


def _body(logits_hbm, tail_hbm, tgt_hbm, mask_hbm, out_hbm,
          buf0, buf1, win_v, tail_v, tgt_v, mask_v, res_v,
          sem0, sem1, semw):
    cid = lax.axis_index("c")
    sid = lax.axis_index("s")
    wid = sid * _NC + cid
    base = wid * _RPW

    pltpu.sync_copy(tgt_hbm.at[pl.ds(pl.multiple_of(base, 8), _RPW)],
                    tgt_v.at[pl.ds(0, _RPW)])
    pltpu.sync_copy(mask_hbm.at[pl.ds(pl.multiple_of(base, 8), _RPW)],
                    mask_v.at[pl.ds(0, _RPW)])
    tgt_all = tgt_v[...]
    mask_all = mask_v[...]

    lane = lax.iota(jnp.int32, 16)
    zeros_i = jnp.zeros((16,), jnp.int32)
    ones_i = jnp.ones((16,), jnp.int32)
    zeros_f = jnp.zeros((16,), jnp.float32)

    def issue(buf, sem, cb, w):
        return pltpu.async_copy(
            logits_hbm.at[pl.ds(pl.multiple_of(base, 8), _RPW), pl.ds(cb, w)],
            buf.at[:, pl.ds(0, w)], sem)

    def wait(buf, sem, cb, w):
        pltpu.make_async_copy(
            logits_hbm.at[pl.ds(pl.multiple_of(base, 8), _RPW), pl.ds(cb, w)],
            buf.at[:, pl.ds(0, w)], sem).wait()

    def process(buf, cb, w, counts):
        nvec = w // 16
        out = []
        for j in range(_RPW):
            vt = vts[j]
            tvec = tvecs[j]
            t = tgt_all[j]
            t_rel = jnp.clip(t - cb, 0, w)
            nge = lax.shift_right_logical(t_rel, 4)

            @plsc.parallel_loop(0, nge, 1, unroll=8, carry=counts[j])
            def ge_loop(i, cnt):
                v = buf[j, pl.ds(pl.multiple_of(i * 16, 16), 16)]
                return cnt + jnp.where(v >= vt, ones_i, zeros_i)

            count = ge_loop
            bidx = jnp.minimum(nge, nvec - 1)
            v = buf[j, pl.ds(pl.multiple_of(bidx * 16, 16), 16)]
            pos = lane + (cb + bidx * 16)
            mixed = (v > vt) | ((v == vt) & (pos < tvec))
            bcnt = jnp.where(mixed, ones_i, zeros_i)
            guard = jnp.full((16,), nge < nvec)
            count = count + jnp.where(guard, bcnt, zeros_i)

            @plsc.parallel_loop(nge + 1, nvec, 1, unroll=8, carry=count)
            def gt_loop(i, cnt):
                v = buf[j, pl.ds(pl.multiple_of(i * 16, 16), 16)]
                return cnt + jnp.where(v > vt, ones_i, zeros_i)

            out.append(gt_loop)
        return tuple(out)

    issue(buf0, sem0, 0, _W)
    issue(buf1, sem1, _W, _W)

    tail_cp = pltpu.async_copy(
        tail_hbm.at[pl.ds(pl.multiple_of(base, 8), _RPW), :], tail_v, semw)
    win_cps = []
    for j in range(_RPW):
        t = tgt_all[j]
        ws = pl.multiple_of(jnp.minimum(t & -128, _V - 160), 128)
        win_cps.append(pltpu.async_copy(
            logits_hbm.at[pl.ds(pl.multiple_of(base, 8), _RPW),
                          pl.ds(ws, 128)],
            win_v.at[pl.ds(j * _RPW, _RPW), :], semw))
    tail_cp.wait()
    for cp in win_cps:
        cp.wait()

    vts = []
    tvecs = []
    for j in range(_RPW):
        t = tgt_all[j]
        tvecs.append(jnp.full((16,), t, jnp.int32))
        ws = jnp.minimum(t & -128, _V - 160)
        colw = jnp.minimum(t - ws, 127)
        vw = plsc.load_gather(
            win_v, [jnp.full((16,), j * _RPW + j, jnp.int32),
                    jnp.full((16,), colw, jnp.int32)])
        colt = jnp.clip(t - _TVO, 0, 127)
        vt_tail = plsc.load_gather(
            tail_v, [jnp.full((16,), j, jnp.int32),
                     jnp.full((16,), colt, jnp.int32)])
        in_tail = jnp.full((16,), t >= _TAIL0)
        vts.append(jnp.where(in_tail, vt_tail, vw))


    counts0 = tuple(zeros_i for _ in range(_RPW))

    def pair_step(g, counts):
        cb = g * (2 * _W)
        wait(buf0, sem0, cb, _W)
        counts = process(buf0, cb, _W, counts)

        @pl.when(g <= _NMAIN // 2 - 1)
        def _():
            issue(buf0, sem0, cb + 2 * _W, _W)

        wait(buf1, sem1, cb + _W, _W)
        counts = process(buf1, cb + _W, _W, counts)

        @pl.when(g <= _NMAIN // 2 - 2)
        def _():
            issue(buf1, sem1, cb + 3 * _W, _W)

        @pl.when(g == _NMAIN // 2 - 1)
        def _():
            issue(buf1, sem1, _CBR, _WR)

        return counts

    counts0 = lax.fori_loop(0, _NMAIN // 2, pair_step, counts0)

    cb14 = (_NMAIN - 1) * _W
    wait(buf0, sem0, cb14, _W)
    counts0 = process(buf0, cb14, _W, counts0)
    wait(buf1, sem1, _CBR, _WR)
    counts0 = process(buf1, _CBR, _WR, counts0)

    counts = list(counts0)
    for j in range(_RPW):
        vt = vts[j]
        tvec = tvecs[j]
        for k in range(2):
            v = tail_v[j, pl.ds(96 + k * 16, 16)]
            pos = lane + (_TAIL0 + k * 16)
            mixed = (v > vt) | ((v == vt) & (pos < tvec))
            counts[j] = counts[j] + jnp.where(mixed, ones_i, zeros_i)

    res = zeros_f
    for j in range(_RPW):
        rank = jnp.sum(counts[j])
        mf = mask_all[j].astype(jnp.float32)
        hitm = jnp.where(rank < _ACC_K, mf, jnp.float32(0.0))
        res = res + jnp.where(lane == j, jnp.full((16,), hitm), zeros_f)
        res = res + jnp.where(lane == (8 + j), jnp.full((16,), mf), zeros_f)

    res_v[...] = res
    pltpu.sync_copy(res_v, out_hbm.at[pl.ds(wid * 16, 16)])


_sc_call = pl.kernel(
    _body,
    out_type=jax.ShapeDtypeStruct((_NW * 16,), jnp.float32),
    mesh=plsc.VectorSubcoreMesh(core_axis_name="c", subcore_axis_name="s"),
    compiler_params=pltpu.CompilerParams(needs_layout_passes=False),
    scratch_types=[
        pltpu.VMEM((_RPW, _W), jnp.float32),
        pltpu.VMEM((_RPW, _W), jnp.float32),
        pltpu.VMEM((_RPW * _RPW, 128), jnp.float32),
        pltpu.VMEM((_RPW, 128), jnp.float32),
        pltpu.VMEM((16,), jnp.int32),
        pltpu.VMEM((16,), jnp.int32),
        pltpu.VMEM((16,), jnp.float32),
        pltpu.SemaphoreType.DMA,
        pltpu.SemaphoreType.DMA,
        pltpu.SemaphoreType.DMA,
    ],
)


def kernel(logits, target, mask):
    logits2d = logits.reshape(_N, _V)
    tail = logits2d[:, _TVO:]
    tgt = target.reshape(-1).astype(jnp.int32)
    msk = mask.reshape(-1).astype(jnp.int32)
    part = _sc_call(logits2d, tail, tgt, msk).reshape(_NW, 16)
    counter = jnp.sum(part[:, :_RPW])
    all_counter = jnp.sum(part[:, _RPW:])
    return (counter / all_counter)[None].astype(jnp.float32)
